# baseline (speedup 1.0000x reference)
import functools

import jax
import jax.numpy as jnp
from jax import lax
from jax.experimental import pallas as pl
from jax.experimental.pallas import tpu as pltpu
from jax.experimental.pallas import tpu_sc as plsc

NC = 2
NS = 16
NW = NC * NS
CHUNK = 128
ZCH = 128
LANES = 16


def _round_up(v, m):
    return (v + m - 1) // m * m


def _sc_degree(edge3, acc_rows):
    nch = edge3.shape[1]
    base = nch // NW
    extra = nch % NW
    rpt = acc_rows // NS
    group = max(d for d in range(1, 9) if base % d == 0)
    mesh = plsc.VectorSubcoreMesh(core_axis_name="c", subcore_axis_name="s")

    @functools.partial(
        pl.kernel,
        out_type=jax.ShapeDtypeStruct((NC, acc_rows, LANES), jnp.float32),
        mesh=mesh,
        scratch_types=[
            pltpu.VMEM((base + 1, CHUNK), jnp.int32),
            pltpu.VMEM((CHUNK, LANES), jnp.float32),
            pltpu.VMEM((ZCH, LANES), jnp.float32),
            pltpu.VMEM_SHARED((acc_rows, LANES), jnp.float32),
            pltpu.SemaphoreType.DMA,
        ],
        compiler_params=pltpu.CompilerParams(use_tc_tiling_on_sc=False),
    )
    def deg_kernel(edge_hbm, deg_out, dstbuf, ones_v, zeros_v, deg_s, sem):
        c = lax.axis_index("c")
        s = lax.axis_index("s")
        wid = c * NS + s

        def fill(i, _):
            ones_v[i, :] = jnp.ones((LANES,), jnp.float32)
            return 0

        lax.fori_loop(0, CHUNK, fill, 0)

        def fillz(i, _):
            zeros_v[i, :] = jnp.zeros((LANES,), jnp.float32)
            return 0

        lax.fori_loop(0, ZCH, fillz, 0)

        def zero_slab(r, _):
            pltpu.sync_copy(
                zeros_v, deg_s.at[pl.ds(s * rpt + r * ZCH, ZCH)]
            )
            return 0

        lax.fori_loop(0, rpt // ZCH, zero_slab, 0)
        plsc.subcore_barrier()

        pltpu.sync_copy(edge_hbm.at[1, pl.ds(wid * base, base)],
                        dstbuf.at[pl.ds(0, base)])

        @pl.when(wid < extra)
        def _():
            pltpu.sync_copy(edge_hbm.at[1, pl.ds(NW * base + wid, 1)],
                            dstbuf.at[pl.ds(base, 1)])

        def grp(g, _):
            def fire(j, _):
                pltpu.async_copy(ones_v, deg_s.at[dstbuf.at[j]], sem,
                                 add=True)
                return 0

            lax.fori_loop(g * group, (g + 1) * group, fire, 0)

            def drain(j, _):
                pltpu.make_async_copy(
                    ones_v, deg_s.at[dstbuf.at[j]], sem).wait()
                return 0

            lax.fori_loop(g * group, (g + 1) * group, drain, 0)
            return 0

        lax.fori_loop(0, base // group, grp, 0)

        @pl.when(wid < extra)
        def _():
            pltpu.sync_copy(ones_v, deg_s.at[dstbuf.at[base]], add=True)

        plsc.subcore_barrier()

        pltpu.sync_copy(
            deg_s.at[pl.ds(s * rpt, rpt)],
            deg_out.at[c, pl.ds(s * rpt, rpt)],
        )

    return deg_kernel(edge3)


def _sc_messages(y, edge3, acc_rows, out_ch):
    nch = edge3.shape[1]
    base = nch // NW
    extra = nch % NW
    rpt = acc_rows // NS
    K = 2
    nsuper = base // K
    assert base % K == 0 and nsuper % 3 == 0 and nsuper >= 6
    mesh = plsc.VectorSubcoreMesh(core_axis_name="c", subcore_axis_name="s")

    @functools.partial(
        pl.kernel,
        out_type=jax.ShapeDtypeStruct((NC, acc_rows, out_ch), jnp.float32),
        mesh=mesh,
        scratch_types=[
            pltpu.VMEM((base + 1, CHUNK), jnp.int32),
            pltpu.VMEM((base + 1, CHUNK), jnp.int32),
            pltpu.VMEM((K * CHUNK, out_ch), jnp.float32),
            pltpu.VMEM((K * CHUNK, out_ch), jnp.float32),
            pltpu.VMEM((K * CHUNK, out_ch), jnp.float32),
            pltpu.VMEM((ZCH, out_ch), jnp.float32),
            pltpu.VMEM_SHARED((acc_rows, out_ch), jnp.float32),
            pltpu.SemaphoreType.DMA,
            pltpu.SemaphoreType.DMA,
            pltpu.SemaphoreType.DMA,
            pltpu.SemaphoreType.DMA,
            pltpu.SemaphoreType.DMA,
            pltpu.SemaphoreType.DMA,
        ],
        compiler_params=pltpu.CompilerParams(use_tc_tiling_on_sc=False),
    )
    def msg_kernel(y_hbm, edge_hbm, acc_out,
                   srcbuf, dstbuf, rows_a, rows_b, rows_c, zeros_v, acc_s,
                   sem_ga, sem_gb, sem_gc, sem_sa, sem_sb, sem_sc):
        c = lax.axis_index("c")
        s = lax.axis_index("s")
        wid = c * NS + s
        lanes_per_row = out_ch // LANES

        def fill(t, _):
            zeros_v[t // lanes_per_row,
                    pl.ds((t % lanes_per_row) * LANES, LANES)] = (
                jnp.zeros((LANES,), jnp.float32))
            return 0

        lax.fori_loop(0, ZCH * lanes_per_row, fill, 0)

        def zero_slab(r, _):
            pltpu.sync_copy(
                zeros_v, acc_s.at[pl.ds(s * rpt + r * ZCH, ZCH)]
            )
            return 0

        lax.fori_loop(0, rpt // ZCH, zero_slab, 0)
        plsc.subcore_barrier()

        pltpu.sync_copy(edge_hbm.at[0, pl.ds(wid * base, base)],
                        srcbuf.at[pl.ds(0, base)])
        pltpu.sync_copy(edge_hbm.at[1, pl.ds(wid * base, base)],
                        dstbuf.at[pl.ds(0, base)])

        @pl.when(wid < extra)
        def _():
            pltpu.sync_copy(edge_hbm.at[0, pl.ds(NW * base + wid, 1)],
                            srcbuf.at[pl.ds(base, 1)])
            pltpu.sync_copy(edge_hbm.at[1, pl.ds(NW * base + wid, 1)],
                            dstbuf.at[pl.ds(base, 1)])

        def fire_gathers(js, buf, sem):
            for i in range(K):
                pltpu.async_copy(
                    y_hbm.at[srcbuf.at[js * K + i]],
                    buf.at[pl.ds(i * CHUNK, CHUNK)], sem)

        def drain_gathers(js, buf, sem):
            for i in range(K):
                pltpu.make_async_copy(
                    y_hbm.at[srcbuf.at[js * K + i]],
                    buf.at[pl.ds(i * CHUNK, CHUNK)], sem).wait()

        def fire_scatters(js, buf, sem):
            for i in range(K):
                pltpu.async_copy(
                    buf.at[pl.ds(i * CHUNK, CHUNK)],
                    acc_s.at[dstbuf.at[js * K + i]], sem, add=True)

        def drain_scatters(js, buf, sem):
            for i in range(K):
                pltpu.make_async_copy(
                    buf.at[pl.ds(i * CHUNK, CHUNK)],
                    acc_s.at[dstbuf.at[js * K + i]], sem).wait()

        ring = ((rows_a, sem_ga, sem_sa),
                (rows_b, sem_gb, sem_sb),
                (rows_c, sem_gc, sem_sc))

        def step(j, cur, prev, drain_prev=True, fire_next=True):
            drain_gathers(j, cur[0], cur[1])
            fire_scatters(j, cur[0], cur[2])
            if drain_prev:
                drain_scatters(j - 1, prev[0], prev[2])
            if fire_next:
                fire_gathers(j + 2, prev[0], prev[1])

        fire_gathers(0, rows_a, sem_ga)
        fire_gathers(1, rows_b, sem_gb)
        step(0, ring[0], ring[2], drain_prev=False)
        step(1, ring[1], ring[0])
        step(2, ring[2], ring[1])

        def triple(g, _):
            j0 = 3 * g
            step(j0, ring[0], ring[2])
            step(j0 + 1, ring[1], ring[0])
            step(j0 + 2, ring[2], ring[1])
            return 0

        lax.fori_loop(1, nsuper // 3 - 1, triple, 0)
        j0 = nsuper - 3
        step(j0, ring[0], ring[2])
        step(j0 + 1, ring[1], ring[0], fire_next=False)
        step(j0 + 2, ring[2], ring[1], fire_next=False)
        drain_scatters(nsuper - 1, ring[2][0], ring[2][2])

        @pl.when(wid < extra)
        def _():
            pltpu.async_copy(
                y_hbm.at[srcbuf.at[base]],
                rows_a.at[pl.ds(0, CHUNK)], sem_ga).wait()
            pltpu.sync_copy(rows_a.at[pl.ds(0, CHUNK)],
                            acc_s.at[dstbuf.at[base]], add=True)

        plsc.subcore_barrier()

        pltpu.sync_copy(
            acc_s.at[pl.ds(s * rpt, rpt)],
            acc_out.at[c, pl.ds(s * rpt, rpt)],
        )

    return msg_kernel(y, edge3)


def _tc_matmul(x2, w2):
    half = x2.shape[0]

    def body(x_ref, w_ref, xw_ref):
        xw_ref[...] = jnp.dot(x_ref[...], w_ref[...],
                              preferred_element_type=jnp.float32)

    return pl.pallas_call(
        body,
        out_shape=jax.ShapeDtypeStruct((half, 128), jnp.float32),
    )(x2, w2)


def _tc_scale(xw128, degp, sel, n):
    half = xw128.shape[0]
    nr = n // 8

    def body(xw_ref, deg_ref, sel_ref, y_ref):
        dinv = _packed_dinv(deg_ref, sel_ref, n)
        y_ref[...] = xw_ref[...] * dinv

    return pl.pallas_call(
        body,
        out_shape=jax.ShapeDtypeStruct((half, 128), jnp.float32),
    )(xw128, degp, sel)


def _packed_dinv(deg_ref, sel_ref, n):
    nr = n // 8
    a = deg_ref[0, :nr, :] + deg_ref[1, :nr, :]
    parts = [
        jnp.dot(a, sel_ref[k], preferred_element_type=jnp.float32)
        [:, None, :]
        for k in range(4)
    ]
    degsel = jnp.concatenate(parts, axis=1).reshape(n // 2, 128)
    return 1.0 / jnp.sqrt(degsel + 1.0)


def _tc_final(acc2, y128, degp, sel, b128, n, out_ch):
    half = n * out_ch // 128

    def body(acc_ref, y_ref, deg_ref, sel_ref, b_ref, o_ref):
        dinv = _packed_dinv(deg_ref, sel_ref, n)
        p = acc_ref[0, :half, :] + acc_ref[1, :half, :] + y_ref[...]
        o_ref[...] = p * dinv + b_ref[...]

    return pl.pallas_call(
        body,
        out_shape=jax.ShapeDtypeStruct((half, 128), jnp.float32),
    )(acc2, y128, degp, sel, b128)


def kernel(x, edge_index, W, b):
    n = x.shape[0]
    out_ch = W.shape[1]
    e = edge_index.shape[1]
    assert e % CHUNK == 0

    acc_rows = _round_up(n, NS * CHUNK)
    nch = e // CHUNK
    edge3 = edge_index.reshape(2, nch, CHUNK)

    deg_part = _sc_degree(edge3, acc_rows)
    degp = deg_part.reshape(NC, acc_rows * LANES // 128, 128)

    in_ch = x.shape[1]
    x2 = x.reshape(n // 2, 2 * in_ch)
    z = jnp.zeros_like(W)
    w2 = jnp.concatenate(
        [jnp.concatenate([W, z], axis=1), jnp.concatenate([z, W], axis=1)],
        axis=0)
    xw128 = _tc_matmul(x2, w2)

    lane = lax.broadcasted_iota(jnp.int32, (128, 128), 1)
    row = lax.broadcasted_iota(jnp.int32, (128, 128), 0)
    sel = jnp.stack([
        (row == jnp.where(lane < out_ch, 32 * k, 32 * k + LANES))
        .astype(jnp.float32)
        for k in range(4)])

    y128 = _tc_scale(xw128, degp, sel, n)
    y = y128.reshape(n, out_ch)
    acc_part = _sc_messages(y, edge3, acc_rows, out_ch)
    acc2 = acc_part.reshape(NC, acc_rows * out_ch // 128, 128)
    b128 = jnp.tile(b, 128 // out_ch).reshape(1, 128)
    out128 = _tc_final(acc2, y128, degp, sel, b128, n, out_ch)
    return out128.reshape(n, out_ch)


# speedup vs baseline: 1.0255x; 1.0255x over previous
import functools

import jax
import jax.numpy as jnp
from jax import lax
from jax.experimental import pallas as pl
from jax.experimental.pallas import tpu as pltpu
from jax.experimental.pallas import tpu_sc as plsc

NC = 2
NS = 16
NW = NC * NS
CHUNK = 128
ZCH = 128
LANES = 16


def _round_up(v, m):
    return (v + m - 1) // m * m


def _sc_degree(edge3, acc_rows):
    nch = edge3.shape[1]
    base = nch // NW
    extra = nch % NW
    rpt = acc_rows // NS
    group = max(d for d in range(1, 9) if base % d == 0)
    mesh = plsc.VectorSubcoreMesh(core_axis_name="c", subcore_axis_name="s")

    @functools.partial(
        pl.kernel,
        out_type=jax.ShapeDtypeStruct((NC, acc_rows, LANES), jnp.float32),
        mesh=mesh,
        scratch_types=[
            pltpu.VMEM((base + 1, CHUNK), jnp.int32),
            pltpu.VMEM((CHUNK, LANES), jnp.float32),
            pltpu.VMEM((ZCH, LANES), jnp.float32),
            pltpu.VMEM_SHARED((acc_rows, LANES), jnp.float32),
            pltpu.SemaphoreType.DMA,
        ],
        compiler_params=pltpu.CompilerParams(use_tc_tiling_on_sc=False),
    )
    def deg_kernel(edge_hbm, deg_out, dstbuf, ones_v, zeros_v, deg_s, sem):
        c = lax.axis_index("c")
        s = lax.axis_index("s")
        wid = c * NS + s

        def fill(i, _):
            ones_v[i, :] = jnp.ones((LANES,), jnp.float32)
            return 0

        lax.fori_loop(0, CHUNK, fill, 0)

        def fillz(i, _):
            zeros_v[i, :] = jnp.zeros((LANES,), jnp.float32)
            return 0

        lax.fori_loop(0, ZCH, fillz, 0)

        def zero_slab(r, _):
            pltpu.sync_copy(
                zeros_v, deg_s.at[pl.ds(s * rpt + r * ZCH, ZCH)]
            )
            return 0

        lax.fori_loop(0, rpt // ZCH, zero_slab, 0)
        plsc.subcore_barrier()

        pltpu.sync_copy(edge_hbm.at[1, pl.ds(wid * base, base)],
                        dstbuf.at[pl.ds(0, base)])

        @pl.when(wid < extra)
        def _():
            pltpu.sync_copy(edge_hbm.at[1, pl.ds(NW * base + wid, 1)],
                            dstbuf.at[pl.ds(base, 1)])

        def grp(g, _):
            def fire(j, _):
                pltpu.async_copy(ones_v, deg_s.at[dstbuf.at[j]], sem,
                                 add=True)
                return 0

            lax.fori_loop(g * group, (g + 1) * group, fire, 0)

            def drain(j, _):
                pltpu.make_async_copy(
                    ones_v, deg_s.at[dstbuf.at[j]], sem).wait()
                return 0

            lax.fori_loop(g * group, (g + 1) * group, drain, 0)
            return 0

        lax.fori_loop(0, base // group, grp, 0)

        @pl.when(wid < extra)
        def _():
            pltpu.sync_copy(ones_v, deg_s.at[dstbuf.at[base]], add=True)

        plsc.subcore_barrier()

        pltpu.sync_copy(
            deg_s.at[pl.ds(s * rpt, rpt)],
            deg_out.at[c, pl.ds(s * rpt, rpt)],
        )

    return deg_kernel(edge3)


def _sc_messages(y, edge3, acc_rows, out_ch):
    nch = edge3.shape[1]
    base = nch // NW
    extra = nch % NW
    rpt = acc_rows // NS
    K = 2
    nsuper = base // K
    assert base % K == 0 and nsuper % 3 == 0 and nsuper >= 6
    mesh = plsc.VectorSubcoreMesh(core_axis_name="c", subcore_axis_name="s")

    @functools.partial(
        pl.kernel,
        out_type=jax.ShapeDtypeStruct((NC, acc_rows, out_ch), jnp.float32),
        mesh=mesh,
        scratch_types=[
            pltpu.VMEM((base + 1, CHUNK), jnp.int32),
            pltpu.VMEM((base + 1, CHUNK), jnp.int32),
            pltpu.VMEM((K * CHUNK, out_ch), jnp.float32),
            pltpu.VMEM((K * CHUNK, out_ch), jnp.float32),
            pltpu.VMEM((K * CHUNK, out_ch), jnp.float32),
            pltpu.VMEM((ZCH, out_ch), jnp.float32),
            pltpu.VMEM_SHARED((acc_rows, out_ch), jnp.float32),
            pltpu.SemaphoreType.DMA,
            pltpu.SemaphoreType.DMA,
            pltpu.SemaphoreType.DMA,
            pltpu.SemaphoreType.DMA,
            pltpu.SemaphoreType.DMA,
            pltpu.SemaphoreType.DMA,
        ],
        compiler_params=pltpu.CompilerParams(use_tc_tiling_on_sc=False),
    )
    def msg_kernel(y_hbm, edge_hbm, acc_out,
                   srcbuf, dstbuf, rows_a, rows_b, rows_c, zeros_v, acc_s,
                   sem_ga, sem_gb, sem_gc, sem_sa, sem_sb, sem_sc):
        c = lax.axis_index("c")
        s = lax.axis_index("s")
        wid = c * NS + s
        lanes_per_row = out_ch // LANES

        def fill(t, _):
            zeros_v[t // lanes_per_row,
                    pl.ds((t % lanes_per_row) * LANES, LANES)] = (
                jnp.zeros((LANES,), jnp.float32))
            return 0

        lax.fori_loop(0, ZCH * lanes_per_row, fill, 0)

        def zero_slab(r, _):
            pltpu.sync_copy(
                zeros_v, acc_s.at[pl.ds(s * rpt + r * ZCH, ZCH)]
            )
            return 0

        lax.fori_loop(0, rpt // ZCH, zero_slab, 0)
        plsc.subcore_barrier()

        pltpu.sync_copy(edge_hbm.at[0, pl.ds(wid * base, base)],
                        srcbuf.at[pl.ds(0, base)])
        pltpu.sync_copy(edge_hbm.at[1, pl.ds(wid * base, base)],
                        dstbuf.at[pl.ds(0, base)])

        @pl.when(wid < extra)
        def _():
            pltpu.sync_copy(edge_hbm.at[0, pl.ds(NW * base + wid, 1)],
                            srcbuf.at[pl.ds(base, 1)])
            pltpu.sync_copy(edge_hbm.at[1, pl.ds(NW * base + wid, 1)],
                            dstbuf.at[pl.ds(base, 1)])

        def fire_gathers(js, buf, sem):
            for i in range(K):
                pltpu.async_copy(
                    y_hbm.at[srcbuf.at[js * K + i]],
                    buf.at[pl.ds(i * CHUNK, CHUNK)], sem)

        def drain_gathers(js, buf, sem):
            for i in range(K):
                pltpu.make_async_copy(
                    y_hbm.at[srcbuf.at[js * K + i]],
                    buf.at[pl.ds(i * CHUNK, CHUNK)], sem).wait()

        def fire_scatters(js, buf, sem):
            for i in range(K):
                pltpu.async_copy(
                    buf.at[pl.ds(i * CHUNK, CHUNK)],
                    acc_s.at[dstbuf.at[js * K + i]], sem, add=True)

        def drain_scatters(js, buf, sem):
            for i in range(K):
                pltpu.make_async_copy(
                    buf.at[pl.ds(i * CHUNK, CHUNK)],
                    acc_s.at[dstbuf.at[js * K + i]], sem).wait()

        ring = ((rows_a, sem_ga, sem_sa),
                (rows_b, sem_gb, sem_sb),
                (rows_c, sem_gc, sem_sc))

        def step(j, cur, prev, drain_prev=True, fire_next=True):
            drain_gathers(j, cur[0], cur[1])
            fire_scatters(j, cur[0], cur[2])
            if drain_prev:
                drain_scatters(j - 1, prev[0], prev[2])
            if fire_next:
                fire_gathers(j + 2, prev[0], prev[1])

        fire_gathers(0, rows_a, sem_ga)
        fire_gathers(1, rows_b, sem_gb)
        step(0, ring[0], ring[2], drain_prev=False)
        step(1, ring[1], ring[0])
        step(2, ring[2], ring[1])

        def triple(g, _):
            j0 = 3 * g
            step(j0, ring[0], ring[2])
            step(j0 + 1, ring[1], ring[0])
            step(j0 + 2, ring[2], ring[1])
            return 0

        lax.fori_loop(1, nsuper // 3 - 1, triple, 0)
        j0 = nsuper - 3
        step(j0, ring[0], ring[2])
        step(j0 + 1, ring[1], ring[0], fire_next=False)
        step(j0 + 2, ring[2], ring[1], fire_next=False)
        drain_scatters(nsuper - 1, ring[2][0], ring[2][2])

        @pl.when(wid < extra)
        def _():
            pltpu.async_copy(
                y_hbm.at[srcbuf.at[base]],
                rows_a.at[pl.ds(0, CHUNK)], sem_ga).wait()
            pltpu.sync_copy(rows_a.at[pl.ds(0, CHUNK)],
                            acc_s.at[dstbuf.at[base]], add=True)

        plsc.subcore_barrier()

        pltpu.sync_copy(
            acc_s.at[pl.ds(s * rpt, rpt)],
            acc_out.at[c, pl.ds(s * rpt, rpt)],
        )

    return msg_kernel(y, edge3)


def _tc_matmul(x2, w2):
    half = x2.shape[0]

    def body(x_ref, w_ref, xw_ref):
        xw_ref[...] = jnp.dot(x_ref[...], w_ref[...],
                              preferred_element_type=jnp.float32)

    return pl.pallas_call(
        body,
        out_shape=jax.ShapeDtypeStruct((half, 128), jnp.float32),
    )(x2, w2)


def _tc_scale(xw128, degp, sel, n):
    half = xw128.shape[0]
    nr = n // 8

    def body(xw_ref, deg_ref, sel_ref, y_ref, dinv_ref):
        dinv = _packed_dinv(deg_ref, sel_ref, n)
        y_ref[...] = xw_ref[...] * dinv
        dinv_ref[...] = dinv

    return pl.pallas_call(
        body,
        out_shape=[
            jax.ShapeDtypeStruct((half, 128), jnp.float32),
            jax.ShapeDtypeStruct((half, 128), jnp.float32),
        ],
    )(xw128, degp, sel)


def _packed_dinv(deg_ref, sel_ref, n):
    nr = n // 8
    a = deg_ref[0, :nr, :] + deg_ref[1, :nr, :]
    parts = [
        jnp.dot(a, sel_ref[k], preferred_element_type=jnp.float32)
        [:, None, :]
        for k in range(4)
    ]
    degsel = jnp.concatenate(parts, axis=1).reshape(n // 2, 128)
    return 1.0 / jnp.sqrt(degsel + 1.0)


def _tc_final(acc2, y128, dinv128, b128, n, out_ch):
    half = n * out_ch // 128

    def body(acc_ref, y_ref, dinv_ref, b_ref, o_ref):
        p = acc_ref[0, :half, :] + acc_ref[1, :half, :] + y_ref[...]
        o_ref[...] = p * dinv_ref[...] + b_ref[...]

    return pl.pallas_call(
        body,
        out_shape=jax.ShapeDtypeStruct((half, 128), jnp.float32),
    )(acc2, y128, dinv128, b128)


def kernel(x, edge_index, W, b):
    n = x.shape[0]
    out_ch = W.shape[1]
    e = edge_index.shape[1]
    assert e % CHUNK == 0

    acc_rows = _round_up(n, NS * CHUNK)
    nch = e // CHUNK
    edge3 = edge_index.reshape(2, nch, CHUNK)

    deg_part = _sc_degree(edge3, acc_rows)
    degp = deg_part.reshape(NC, acc_rows * LANES // 128, 128)

    in_ch = x.shape[1]
    x2 = x.reshape(n // 2, 2 * in_ch)
    z = jnp.zeros_like(W)
    w2 = jnp.concatenate(
        [jnp.concatenate([W, z], axis=1), jnp.concatenate([z, W], axis=1)],
        axis=0)
    xw128 = _tc_matmul(x2, w2)

    lane = lax.broadcasted_iota(jnp.int32, (128, 128), 1)
    row = lax.broadcasted_iota(jnp.int32, (128, 128), 0)
    sel = jnp.stack([
        (row == jnp.where(lane < out_ch, 32 * k, 32 * k + LANES))
        .astype(jnp.float32)
        for k in range(4)])

    y128, dinv128 = _tc_scale(xw128, degp, sel, n)
    y = y128.reshape(n, out_ch)
    acc_part = _sc_messages(y, edge3, acc_rows, out_ch)
    acc2 = acc_part.reshape(NC, acc_rows * out_ch // 128, 128)
    b128 = jnp.tile(b, 128 // out_ch).reshape(1, 128)
    out128 = _tc_final(acc2, y128, dinv128, b128, n, out_ch)
    return out128.reshape(n, out_ch)


# speedup vs baseline: 1.0699x; 1.0432x over previous
import functools

import jax
import jax.numpy as jnp
from jax import lax
from jax.experimental import pallas as pl
from jax.experimental.pallas import tpu as pltpu
from jax.experimental.pallas import tpu_sc as plsc

NC = 2
NS = 16
NW = NC * NS
CHUNK = 128
ZCH = 128
LANES = 16


def _round_up(v, m):
    return (v + m - 1) // m * m


def _sc_degree(edge3, acc_rows):
    nch = edge3.shape[1]
    base = nch // NW
    extra = nch % NW
    rpt = acc_rows // NS
    group = max(d for d in range(1, 9) if base % d == 0)
    mesh = plsc.VectorSubcoreMesh(core_axis_name="c", subcore_axis_name="s")

    @functools.partial(
        pl.kernel,
        out_type=jax.ShapeDtypeStruct((NC, acc_rows, LANES), jnp.float32),
        mesh=mesh,
        scratch_types=[
            pltpu.VMEM((base + 1, CHUNK), jnp.int32),
            pltpu.VMEM((CHUNK, LANES), jnp.float32),
            pltpu.VMEM((ZCH, LANES), jnp.float32),
            pltpu.VMEM_SHARED((acc_rows, LANES), jnp.float32),
            pltpu.SemaphoreType.DMA,
            pltpu.SemaphoreType.DMA,
        ],
        compiler_params=pltpu.CompilerParams(use_tc_tiling_on_sc=False),
    )
    def deg_kernel(edge_hbm, deg_out, dstbuf, ones_v, zeros_v, deg_s,
                   sem, sem_idx):
        c = lax.axis_index("c")
        s = lax.axis_index("s")
        wid = c * NS + s

        idx_cp = pltpu.async_copy(edge_hbm.at[1, pl.ds(wid * base, base)],
                                  dstbuf.at[pl.ds(0, base)], sem_idx)

        @pl.when(wid < extra)
        def _():
            pltpu.async_copy(edge_hbm.at[1, pl.ds(NW * base + wid, 1)],
                             dstbuf.at[pl.ds(base, 1)], sem_idx)

        def fill(i, _):
            ones_v[i, :] = jnp.ones((LANES,), jnp.float32)
            return 0

        lax.fori_loop(0, CHUNK, fill, 0)

        def fillz(i, _):
            zeros_v[i, :] = jnp.zeros((LANES,), jnp.float32)
            return 0

        lax.fori_loop(0, ZCH, fillz, 0)

        def zero_slab(r, _):
            pltpu.async_copy(
                zeros_v, deg_s.at[pl.ds(s * rpt + r * ZCH, ZCH)], sem)
            return 0

        lax.fori_loop(0, rpt // ZCH, zero_slab, 0)

        def zero_drain(r, _):
            pltpu.make_async_copy(
                zeros_v, deg_s.at[pl.ds(s * rpt + r * ZCH, ZCH)], sem).wait()
            return 0

        lax.fori_loop(0, rpt // ZCH, zero_drain, 0)
        plsc.subcore_barrier()

        idx_cp.wait()

        @pl.when(wid < extra)
        def _():
            pltpu.make_async_copy(edge_hbm.at[1, pl.ds(NW * base + wid, 1)],
                                  dstbuf.at[pl.ds(base, 1)], sem_idx).wait()

        def grp(g, _):
            def fire(j, _):
                pltpu.async_copy(ones_v, deg_s.at[dstbuf.at[j]], sem,
                                 add=True)
                return 0

            lax.fori_loop(g * group, (g + 1) * group, fire, 0)

            def drain(j, _):
                pltpu.make_async_copy(
                    ones_v, deg_s.at[dstbuf.at[j]], sem).wait()
                return 0

            lax.fori_loop(g * group, (g + 1) * group, drain, 0)
            return 0

        lax.fori_loop(0, base // group, grp, 0)

        @pl.when(wid < extra)
        def _():
            pltpu.sync_copy(ones_v, deg_s.at[dstbuf.at[base]], add=True)

        plsc.subcore_barrier()

        pltpu.sync_copy(
            deg_s.at[pl.ds(s * rpt, rpt)],
            deg_out.at[c, pl.ds(s * rpt, rpt)],
        )

    return deg_kernel(edge3)


def _sc_messages(y, edge3, acc_rows, out_ch):
    nch = edge3.shape[1]
    base = nch // NW
    extra = nch % NW
    rpt = acc_rows // NS
    K = 2
    nsuper = base // K
    assert base % K == 0 and nsuper % 3 == 0 and nsuper >= 6
    mesh = plsc.VectorSubcoreMesh(core_axis_name="c", subcore_axis_name="s")

    @functools.partial(
        pl.kernel,
        out_type=jax.ShapeDtypeStruct((NC, acc_rows, out_ch), jnp.float32),
        mesh=mesh,
        scratch_types=[
            pltpu.VMEM((base + 1, CHUNK), jnp.int32),
            pltpu.VMEM((base + 1, CHUNK), jnp.int32),
            pltpu.VMEM((K * CHUNK, out_ch), jnp.float32),
            pltpu.VMEM((K * CHUNK, out_ch), jnp.float32),
            pltpu.VMEM((K * CHUNK, out_ch), jnp.float32),
            pltpu.VMEM((ZCH, out_ch), jnp.float32),
            pltpu.VMEM_SHARED((acc_rows, out_ch), jnp.float32),
            pltpu.SemaphoreType.DMA,
            pltpu.SemaphoreType.DMA,
            pltpu.SemaphoreType.DMA,
            pltpu.SemaphoreType.DMA,
            pltpu.SemaphoreType.DMA,
            pltpu.SemaphoreType.DMA,
            pltpu.SemaphoreType.DMA,
        ],
        compiler_params=pltpu.CompilerParams(use_tc_tiling_on_sc=False),
    )
    def msg_kernel(y_hbm, edge_hbm, acc_out,
                   srcbuf, dstbuf, rows_a, rows_b, rows_c, zeros_v, acc_s,
                   sem_ga, sem_gb, sem_gc, sem_sa, sem_sb, sem_sc, sem_idx):
        c = lax.axis_index("c")
        s = lax.axis_index("s")
        wid = c * NS + s
        lanes_per_row = out_ch // LANES

        src_cp = pltpu.async_copy(edge_hbm.at[0, pl.ds(wid * base, base)],
                                  srcbuf.at[pl.ds(0, base)], sem_idx)
        dst_cp = pltpu.async_copy(edge_hbm.at[1, pl.ds(wid * base, base)],
                                  dstbuf.at[pl.ds(0, base)], sem_idx)

        @pl.when(wid < extra)
        def _():
            pltpu.async_copy(edge_hbm.at[0, pl.ds(NW * base + wid, 1)],
                             srcbuf.at[pl.ds(base, 1)], sem_idx)
            pltpu.async_copy(edge_hbm.at[1, pl.ds(NW * base + wid, 1)],
                             dstbuf.at[pl.ds(base, 1)], sem_idx)

        def fill(t, _):
            zeros_v[t // lanes_per_row,
                    pl.ds((t % lanes_per_row) * LANES, LANES)] = (
                jnp.zeros((LANES,), jnp.float32))
            return 0

        lax.fori_loop(0, ZCH * lanes_per_row, fill, 0)

        def zero_slab(r, _):
            pltpu.async_copy(
                zeros_v, acc_s.at[pl.ds(s * rpt + r * ZCH, ZCH)], sem_sa)
            return 0

        lax.fori_loop(0, rpt // ZCH, zero_slab, 0)

        def zero_drain(r, _):
            pltpu.make_async_copy(
                zeros_v, acc_s.at[pl.ds(s * rpt + r * ZCH, ZCH)],
                sem_sa).wait()
            return 0

        lax.fori_loop(0, rpt // ZCH, zero_drain, 0)
        plsc.subcore_barrier()

        src_cp.wait()
        dst_cp.wait()

        @pl.when(wid < extra)
        def _():
            pltpu.make_async_copy(edge_hbm.at[0, pl.ds(NW * base + wid, 1)],
                                  srcbuf.at[pl.ds(base, 1)], sem_idx).wait()
            pltpu.make_async_copy(edge_hbm.at[1, pl.ds(NW * base + wid, 1)],
                                  dstbuf.at[pl.ds(base, 1)], sem_idx).wait()

        def fire_gathers(js, buf, sem):
            for i in range(K):
                pltpu.async_copy(
                    y_hbm.at[srcbuf.at[js * K + i]],
                    buf.at[pl.ds(i * CHUNK, CHUNK)], sem)

        def drain_gathers(js, buf, sem):
            for i in range(K):
                pltpu.make_async_copy(
                    y_hbm.at[srcbuf.at[js * K + i]],
                    buf.at[pl.ds(i * CHUNK, CHUNK)], sem).wait()

        def fire_scatters(js, buf, sem):
            for i in range(K):
                pltpu.async_copy(
                    buf.at[pl.ds(i * CHUNK, CHUNK)],
                    acc_s.at[dstbuf.at[js * K + i]], sem, add=True)

        def drain_scatters(js, buf, sem):
            for i in range(K):
                pltpu.make_async_copy(
                    buf.at[pl.ds(i * CHUNK, CHUNK)],
                    acc_s.at[dstbuf.at[js * K + i]], sem).wait()

        ring = ((rows_a, sem_ga, sem_sa),
                (rows_b, sem_gb, sem_sb),
                (rows_c, sem_gc, sem_sc))

        def step(j, cur, prev, drain_prev=True, fire_next=True):
            drain_gathers(j, cur[0], cur[1])
            fire_scatters(j, cur[0], cur[2])
            if drain_prev:
                drain_scatters(j - 1, prev[0], prev[2])
            if fire_next:
                fire_gathers(j + 2, prev[0], prev[1])

        fire_gathers(0, rows_a, sem_ga)
        fire_gathers(1, rows_b, sem_gb)
        step(0, ring[0], ring[2], drain_prev=False)
        step(1, ring[1], ring[0])
        step(2, ring[2], ring[1])

        def triple(g, _):
            j0 = 3 * g
            step(j0, ring[0], ring[2])
            step(j0 + 1, ring[1], ring[0])
            step(j0 + 2, ring[2], ring[1])
            return 0

        lax.fori_loop(1, nsuper // 3 - 1, triple, 0)
        j0 = nsuper - 3
        step(j0, ring[0], ring[2])
        step(j0 + 1, ring[1], ring[0], fire_next=False)
        step(j0 + 2, ring[2], ring[1], fire_next=False)
        drain_scatters(nsuper - 1, ring[2][0], ring[2][2])

        @pl.when(wid < extra)
        def _():
            pltpu.async_copy(
                y_hbm.at[srcbuf.at[base]],
                rows_a.at[pl.ds(0, CHUNK)], sem_ga).wait()
            pltpu.sync_copy(rows_a.at[pl.ds(0, CHUNK)],
                            acc_s.at[dstbuf.at[base]], add=True)

        plsc.subcore_barrier()

        pltpu.sync_copy(
            acc_s.at[pl.ds(s * rpt, rpt)],
            acc_out.at[c, pl.ds(s * rpt, rpt)],
        )

    return msg_kernel(y, edge3)


def _tc_matmul(x2, w2):
    half = x2.shape[0]

    def body(x_ref, w_ref, xw_ref):
        xw_ref[...] = jnp.dot(x_ref[...], w_ref[...],
                              preferred_element_type=jnp.float32)

    return pl.pallas_call(
        body,
        out_shape=jax.ShapeDtypeStruct((half, 128), jnp.float32),
    )(x2, w2)


def _tc_scale(xw128, degp, sel, n):
    half = xw128.shape[0]
    nr = n // 8

    def body(xw_ref, deg_ref, sel_ref, y_ref, dinv_ref):
        dinv = _packed_dinv(deg_ref, sel_ref, n)
        y_ref[...] = xw_ref[...] * dinv
        dinv_ref[...] = dinv

    return pl.pallas_call(
        body,
        out_shape=[
            jax.ShapeDtypeStruct((half, 128), jnp.float32),
            jax.ShapeDtypeStruct((half, 128), jnp.float32),
        ],
    )(xw128, degp, sel)


def _packed_dinv(deg_ref, sel_ref, n):
    nr = n // 8
    a = deg_ref[0, :nr, :] + deg_ref[1, :nr, :]
    parts = [
        jnp.dot(a, sel_ref[k], preferred_element_type=jnp.float32)
        [:, None, :]
        for k in range(4)
    ]
    degsel = jnp.concatenate(parts, axis=1).reshape(n // 2, 128)
    return 1.0 / jnp.sqrt(degsel + 1.0)


def _tc_final(acc2, y128, dinv128, b128, n, out_ch):
    half = n * out_ch // 128

    def body(acc_ref, y_ref, dinv_ref, b_ref, o_ref):
        p = acc_ref[0, :half, :] + acc_ref[1, :half, :] + y_ref[...]
        o_ref[...] = p * dinv_ref[...] + b_ref[...]

    return pl.pallas_call(
        body,
        out_shape=jax.ShapeDtypeStruct((half, 128), jnp.float32),
    )(acc2, y128, dinv128, b128)


def kernel(x, edge_index, W, b):
    n = x.shape[0]
    out_ch = W.shape[1]
    e = edge_index.shape[1]
    assert e % CHUNK == 0

    acc_rows = _round_up(n, NS * CHUNK)
    nch = e // CHUNK
    edge3 = edge_index.reshape(2, nch, CHUNK)

    deg_part = _sc_degree(edge3, acc_rows)
    degp = deg_part.reshape(NC, acc_rows * LANES // 128, 128)

    in_ch = x.shape[1]
    x2 = x.reshape(n // 2, 2 * in_ch)
    z = jnp.zeros_like(W)
    w2 = jnp.concatenate(
        [jnp.concatenate([W, z], axis=1), jnp.concatenate([z, W], axis=1)],
        axis=0)
    xw128 = _tc_matmul(x2, w2)

    lane = lax.broadcasted_iota(jnp.int32, (128, 128), 1)
    row = lax.broadcasted_iota(jnp.int32, (128, 128), 0)
    sel = jnp.stack([
        (row == jnp.where(lane < out_ch, 32 * k, 32 * k + LANES))
        .astype(jnp.float32)
        for k in range(4)])

    y128, dinv128 = _tc_scale(xw128, degp, sel, n)
    y = y128.reshape(n, out_ch)
    acc_part = _sc_messages(y, edge3, acc_rows, out_ch)
    acc2 = acc_part.reshape(NC, acc_rows * out_ch // 128, 128)
    b128 = jnp.tile(b, 128 // out_ch).reshape(1, 128)
    out128 = _tc_final(acc2, y128, dinv128, b128, n, out_ch)
    return out128.reshape(n, out_ch)
